# baseline (speedup 1.0000x reference)
import functools

import jax
import jax.numpy as jnp
from jax import lax
from jax.experimental import pallas as pl
from jax.experimental.pallas import tpu as pltpu
from jax.experimental.pallas import tpu_sc as plsc

_NC, _NS = 2, 16
_NW = _NC * _NS
_H = 64
_B = 16384
_N = 64
_BPW = _B // _NW
_MARGIN = 1.0
_BLK = 2048

_CHUNK = 16
_GBLK = 128


def _fire_chunk(table_hbm, iv, buf_v, off, sem):
    copies = []
    for jj in range(_CHUNK):
        copies.append(pltpu.async_copy(
            table_hbm.at[pl.ds(iv[jj], 1), :],
            buf_v.at[pl.ds(off + jj, 1), :], sem))
    return copies


def _sc_gather(emb, r_emb, pos_h, pos_r, pos_t, neg_h, neg_t, neg_r):
    mesh = plsc.VectorSubcoreMesh(core_axis_name="c", subcore_axis_name="s")
    out_type = (
        [jax.ShapeDtypeStruct((_B, _H), jnp.float32)] * 3
        + [jax.ShapeDtypeStruct((_N, _H), jnp.float32)] * 3
    )
    scratch_types = [
        pltpu.VMEM((_BPW,), jnp.int32),
        pltpu.VMEM((_BPW,), jnp.int32),
        pltpu.VMEM((_BPW,), jnp.int32),
        pltpu.VMEM((_GBLK, _H), jnp.float32),
        pltpu.VMEM((_GBLK, _H), jnp.float32),
        pltpu.VMEM((_GBLK, _H), jnp.float32),
        pltpu.VMEM((_N,), jnp.int32),
        pltpu.VMEM((_N, _H), jnp.float32),
        pltpu.SemaphoreType.DMA,
        pltpu.SemaphoreType.DMA,
        pltpu.SemaphoreType.DMA,
    ]

    @functools.partial(pl.kernel, mesh=mesh, out_type=out_type,
                       scratch_types=scratch_types,
                       compiler_params=pltpu.CompilerParams(
                           use_tc_tiling_on_sc=True))
    def k(emb_hbm, remb_hbm, ph_hbm, pr_hbm, pt_hbm, nh_hbm, nt_hbm, nr_hbm,
          oh, orr, ot, onh, ont, onr,
          ih_v, ir_v, it_v, bh_v, br_v, bt_v, ni_v, nrow_v,
          sem0, sem1, sem2):
        wid = lax.axis_index("s") * _NC + lax.axis_index("c")
        base = wid * _BPW
        sl = pl.ds(base, _BPW)
        pltpu.sync_copy(ph_hbm.at[sl], ih_v)
        pltpu.sync_copy(pr_hbm.at[sl], ir_v)
        pltpu.sync_copy(pt_hbm.at[sl], it_v)

        def blk_body(b):
            def chunk_body(c):
                o = c * _CHUNK
                g = b * _GBLK + o
                iv_h = ih_v[pl.ds(g, _CHUNK)]
                iv_r = ir_v[pl.ds(g, _CHUNK)]
                iv_t = it_v[pl.ds(g, _CHUNK)]
                cs = (_fire_chunk(emb_hbm, iv_h, bh_v, o, sem0)
                      + _fire_chunk(remb_hbm, iv_r, br_v, o, sem1)
                      + _fire_chunk(emb_hbm, iv_t, bt_v, o, sem2))
                for cp in cs:
                    cp.wait()

            pl.loop(0, _GBLK // _CHUNK)(chunk_body)
            out_sl = pl.ds(base + b * _GBLK, _GBLK)
            pltpu.sync_copy(bh_v, oh.at[out_sl])
            pltpu.sync_copy(br_v, orr.at[out_sl])
            pltpu.sync_copy(bt_v, ot.at[out_sl])

        pl.loop(0, _BPW // _GBLK)(blk_body)

        @pl.when(wid == 0)
        def _():
            pltpu.sync_copy(nh_hbm, ni_v)
            for c in range(_N // _CHUNK):
                iv = ni_v[pl.ds(c * _CHUNK, _CHUNK)]
                for cp in _fire_chunk(emb_hbm, iv, nrow_v, c * _CHUNK, sem0):
                    cp.wait()
            pltpu.sync_copy(nrow_v, onh)

        @pl.when(wid == 1)
        def _():
            pltpu.sync_copy(nt_hbm, ni_v)
            for c in range(_N // _CHUNK):
                iv = ni_v[pl.ds(c * _CHUNK, _CHUNK)]
                for cp in _fire_chunk(emb_hbm, iv, nrow_v, c * _CHUNK, sem1):
                    cp.wait()
            pltpu.sync_copy(nrow_v, ont)

        @pl.when(wid == 2)
        def _():
            pltpu.sync_copy(nr_hbm, ni_v)
            for c in range(_N // _CHUNK):
                iv = ni_v[pl.ds(c * _CHUNK, _CHUNK)]
                for cp in _fire_chunk(remb_hbm, iv, nrow_v, c * _CHUNK, sem2):
                    cp.wait()
            pltpu.sync_copy(nrow_v, onr)

    return k(emb, r_emb, pos_h, pos_r, pos_t, neg_h, neg_t, neg_r)


_LOG2E = 1.4426950408889634
_LN2 = 0.6931471805599453


def _softplus_neg_sum(x):
    return jnp.sum(jnp.log2(1.0 + jnp.exp2(x * _LOG2E)))


def _softplus_stable(x):
    return jnp.maximum(x, 0.0) + _LN2 * jnp.log2(
        1.0 + jnp.exp2(-jnp.abs(x) * _LOG2E))


def _tc_body(h_ref, r_ref, t_ref, nh_ref, nt_ref, nr_ref, o_ref):
    i = pl.program_id(0)

    @pl.when(i == 0)
    def _():
        o_ref[...] = jnp.zeros((1, 1), jnp.float32)

    h = h_ref[...]
    r = r_ref[...]
    t = t_ref[...]
    nh = nh_ref[...]
    nt = nt_ref[...]
    nr = nr_ref[...]

    d = h + r - t
    rt = r - t
    hr = h + r
    ht = h - t

    def dot_t(a, b):
        return lax.dot_general(a.astype(jnp.bfloat16), b.astype(jnp.bfloat16),
                               (((1,), (1,)), ((), ())),
                               preferred_element_type=jnp.float32)

    nh2 = 0.5 * jnp.sum(nh * nh, axis=1)
    nt2 = 0.5 * jnp.sum(nt * nt, axis=1)
    nr2 = 0.5 * jnp.sum(nr * nr, axis=1)

    pos = _MARGIN - 0.5 * jnp.sum(d * d, axis=1)
    pos_sum = jnp.sum(_softplus_stable(-pos))

    y_nh = (_MARGIN - 0.5 * jnp.sum(rt * rt, axis=1, keepdims=True)
            - nh2[None, :] - dot_t(rt, nh))
    y_nt = (_MARGIN - 0.5 * jnp.sum(hr * hr, axis=1, keepdims=True)
            - nt2[None, :] + dot_t(hr, nt))
    y_nr = (_MARGIN - 0.5 * jnp.sum(ht * ht, axis=1, keepdims=True)
            - nr2[None, :] - dot_t(ht, nr))

    part = (3.0 * pos_sum
            + _LN2 * (_softplus_neg_sum(y_nh)
                      + _softplus_neg_sum(y_nt)
                      + _softplus_neg_sum(y_nr)))
    o_ref[...] += jnp.full((1, 1), part * (1.0 / _B), jnp.float32)


def _tc_score(h_e, r_e, t_e, nh_e, nt_e, nr_e):
    grid = _B // _BLK
    out = pl.pallas_call(
        _tc_body,
        grid=(grid,),
        in_specs=(
            [pl.BlockSpec((_BLK, _H), lambda i: (i, 0))] * 3
            + [pl.BlockSpec((_N, _H), lambda i: (0, 0))] * 3
        ),
        out_specs=pl.BlockSpec((1, 1), lambda i: (0, 0)),
        out_shape=jax.ShapeDtypeStruct((1, 1), jnp.float32),
    )(h_e, r_e, t_e, nh_e, nt_e, nr_e)
    return out[0, 0]


def kernel(pos_h, pos_r, pos_t, neg_h, neg_t, neg_r, emb, r_emb):
    pos_h = pos_h.astype(jnp.int32)
    pos_r = pos_r.astype(jnp.int32)
    pos_t = pos_t.astype(jnp.int32)
    neg_h = neg_h.astype(jnp.int32)
    neg_t = neg_t.astype(jnp.int32)
    neg_r = neg_r.astype(jnp.int32)
    h_e, r_e, t_e, nh_e, nt_e, nr_e = _sc_gather(
        emb, r_emb, pos_h, pos_r, pos_t, neg_h, neg_t, neg_r)
    loss = _tc_score(h_e, r_e, t_e, nh_e, nt_e, nr_e)
    s = jnp.sum(jnp.take(emb, pos_h, axis=0))
    return loss + (s - s)


# speedup vs baseline: 1.0305x; 1.0305x over previous
import functools

import jax
import jax.numpy as jnp
from jax import lax
from jax.experimental import pallas as pl
from jax.experimental.pallas import tpu as pltpu
from jax.experimental.pallas import tpu_sc as plsc

_NC, _NS = 2, 16
_NW = _NC * _NS
_H = 64
_B = 16384
_N = 64
_BPW = _B // _NW
_MARGIN = 1.0
_BLK = 2048

_CHUNK = 16
_GBLK = 128


def _fire_chunk(table_hbm, iv, buf_v, off, sem):
    copies = []
    for jj in range(_CHUNK):
        copies.append(pltpu.async_copy(
            table_hbm.at[pl.ds(iv[jj], 1), :],
            buf_v.at[pl.ds(off + jj, 1), :], sem))
    return copies


def _sc_gather(emb, r_emb, pos_h, pos_r, pos_t, neg_h, neg_t, neg_r):
    mesh = plsc.VectorSubcoreMesh(core_axis_name="c", subcore_axis_name="s")
    out_type = (
        [jax.ShapeDtypeStruct((_B, _H), jnp.float32)] * 3
        + [jax.ShapeDtypeStruct((_N, _H), jnp.float32)] * 3
    )
    scratch_types = [
        pltpu.VMEM((_BPW,), jnp.int32),
        pltpu.VMEM((_BPW,), jnp.int32),
        pltpu.VMEM((_BPW,), jnp.int32),
        pltpu.VMEM((_GBLK, _H), jnp.float32),
        pltpu.VMEM((_GBLK, _H), jnp.float32),
        pltpu.VMEM((_GBLK, _H), jnp.float32),
        pltpu.VMEM((_N,), jnp.int32),
        pltpu.VMEM((_N, _H), jnp.float32),
        pltpu.SemaphoreType.DMA,
        pltpu.SemaphoreType.DMA,
        pltpu.SemaphoreType.DMA,
    ]

    @functools.partial(pl.kernel, mesh=mesh, out_type=out_type,
                       scratch_types=scratch_types,
                       compiler_params=pltpu.CompilerParams(
                           use_tc_tiling_on_sc=True))
    def k(emb_hbm, remb_hbm, ph_hbm, pr_hbm, pt_hbm, nh_hbm, nt_hbm, nr_hbm,
          oh, orr, ot, onh, ont, onr,
          ih_v, ir_v, it_v, bh_v, br_v, bt_v, ni_v, nrow_v,
          sem0, sem1, sem2):
        wid = lax.axis_index("s") * _NC + lax.axis_index("c")
        base = wid * _BPW
        sl = pl.ds(base, _BPW)
        pltpu.sync_copy(ph_hbm.at[sl], ih_v)
        pltpu.sync_copy(pr_hbm.at[sl], ir_v)
        pltpu.sync_copy(pt_hbm.at[sl], it_v)

        def blk_body(b):
            def chunk_body(c):
                o = c * _CHUNK
                g = b * _GBLK + o
                iv_h = ih_v[pl.ds(g, _CHUNK)]
                iv_r = ir_v[pl.ds(g, _CHUNK)]
                iv_t = it_v[pl.ds(g, _CHUNK)]
                cs = (_fire_chunk(emb_hbm, iv_h, bh_v, o, sem0)
                      + _fire_chunk(remb_hbm, iv_r, br_v, o, sem1)
                      + _fire_chunk(emb_hbm, iv_t, bt_v, o, sem2))
                for cp in cs:
                    cp.wait()

            pl.loop(0, _GBLK // _CHUNK)(chunk_body)
            out_sl = pl.ds(base + b * _GBLK, _GBLK)
            pltpu.sync_copy(bh_v, oh.at[out_sl])
            pltpu.sync_copy(br_v, orr.at[out_sl])
            pltpu.sync_copy(bt_v, ot.at[out_sl])

        pl.loop(0, _BPW // _GBLK)(blk_body)

        @pl.when(wid == 0)
        def _():
            pltpu.sync_copy(nh_hbm, ni_v)
            for c in range(_N // _CHUNK):
                iv = ni_v[pl.ds(c * _CHUNK, _CHUNK)]
                for cp in _fire_chunk(emb_hbm, iv, nrow_v, c * _CHUNK, sem0):
                    cp.wait()
            pltpu.sync_copy(nrow_v, onh)

        @pl.when(wid == 1)
        def _():
            pltpu.sync_copy(nt_hbm, ni_v)
            for c in range(_N // _CHUNK):
                iv = ni_v[pl.ds(c * _CHUNK, _CHUNK)]
                for cp in _fire_chunk(emb_hbm, iv, nrow_v, c * _CHUNK, sem1):
                    cp.wait()
            pltpu.sync_copy(nrow_v, ont)

        @pl.when(wid == 2)
        def _():
            pltpu.sync_copy(nr_hbm, ni_v)
            for c in range(_N // _CHUNK):
                iv = ni_v[pl.ds(c * _CHUNK, _CHUNK)]
                for cp in _fire_chunk(remb_hbm, iv, nrow_v, c * _CHUNK, sem2):
                    cp.wait()
            pltpu.sync_copy(nrow_v, onr)

    return k(emb, r_emb, pos_h, pos_r, pos_t, neg_h, neg_t, neg_r)


_LOG2E = 1.4426950408889634
_LN2 = 0.6931471805599453


def _softplus_neg_sum(x):
    return jnp.sum(jnp.log2(1.0 + jnp.exp2(x * _LOG2E)))


def _softplus_stable(x):
    return jnp.maximum(x, 0.0) + _LN2 * jnp.log2(
        1.0 + jnp.exp2(-jnp.abs(x) * _LOG2E))


def _tc_body(h_ref, r_ref, t_ref, nh_ref, nt_ref, nr_ref, o_ref):
    i = pl.program_id(0)

    @pl.when(i == 0)
    def _():
        o_ref[...] = jnp.zeros((1, 1), jnp.float32)

    h = h_ref[...]
    r = r_ref[...]
    t = t_ref[...]
    nh = nh_ref[...]
    nt = nt_ref[...]
    nr = nr_ref[...]

    d = h + r - t
    rt = r - t
    hr = h + r
    ht = h - t

    def dot_t(a, b):
        return lax.dot_general(a.astype(jnp.bfloat16), b.astype(jnp.bfloat16),
                               (((1,), (1,)), ((), ())),
                               preferred_element_type=jnp.float32)

    nh2 = 0.5 * jnp.sum(nh * nh, axis=1)
    nt2 = 0.5 * jnp.sum(nt * nt, axis=1)
    nr2 = 0.5 * jnp.sum(nr * nr, axis=1)

    pos = _MARGIN - 0.5 * jnp.sum(d * d, axis=1)
    pos_sum = jnp.sum(_softplus_stable(-pos))

    y_nh = (_MARGIN - 0.5 * jnp.sum(rt * rt, axis=1, keepdims=True)
            - nh2[None, :] - dot_t(rt, nh))
    y_nt = (_MARGIN - 0.5 * jnp.sum(hr * hr, axis=1, keepdims=True)
            - nt2[None, :] + dot_t(hr, nt))
    y_nr = (_MARGIN - 0.5 * jnp.sum(ht * ht, axis=1, keepdims=True)
            - nr2[None, :] - dot_t(ht, nr))

    part = (3.0 * pos_sum
            + _LN2 * (_softplus_neg_sum(y_nh)
                      + _softplus_neg_sum(y_nt)
                      + _softplus_neg_sum(y_nr)))
    o_ref[...] += jnp.full((1, 1), part * (1.0 / _B), jnp.float32)


def _tc_score(h_e, r_e, t_e, nh_e, nt_e, nr_e):
    grid = _B // _BLK
    out = pl.pallas_call(
        _tc_body,
        grid=(grid,),
        in_specs=(
            [pl.BlockSpec((_BLK, _H), lambda i: (i, 0))] * 3
            + [pl.BlockSpec((_N, _H), lambda i: (0, 0))] * 3
        ),
        out_specs=pl.BlockSpec((1, 1), lambda i: (0, 0)),
        out_shape=jax.ShapeDtypeStruct((1, 1), jnp.float32),
    )(h_e, r_e, t_e, nh_e, nt_e, nr_e)
    return out[0, 0]


def kernel(pos_h, pos_r, pos_t, neg_h, neg_t, neg_r, emb, r_emb):
    pos_h = pos_h.astype(jnp.int32)
    pos_r = pos_r.astype(jnp.int32)
    pos_t = pos_t.astype(jnp.int32)
    neg_h = neg_h.astype(jnp.int32)
    neg_t = neg_t.astype(jnp.int32)
    neg_r = neg_r.astype(jnp.int32)
    h_e, r_e, t_e, nh_e, nt_e, nr_e = _sc_gather(
        emb, r_emb, pos_h, pos_r, pos_t, neg_h, neg_t, neg_r)
    return _tc_score(h_e, r_e, t_e, nh_e, nt_e, nr_e)
